# baseline (speedup 1.0000x reference)
import jax
import jax.numpy as jnp
from jax import lax
from jax.experimental import pallas as pl
from jax.experimental.pallas import tpu as pltpu
from jax.experimental.pallas import tpu_sc as plsc

_B, _HIS, _D = 64, 50, 256
_N = 100000
_L = 6
_H, _DK = 16, 16
_HID = 200

_NW = 32
_NCH = 16
_CHAINS = _B * _HIS
_NCHUNK = _CHAINS // _NCH
_NPAIR = _NCHUNK // 2
_UNROLL = 4
_BB = 8


def _chain_kernel_body(table, click, nbr, news, x3_out, w8_out,
                       curA, candA, nbrA, clkA, cidxA, idx0A, w8A,
                       curB, candB, nbrB, clkB, cidxB, idx0B, w8B,
                       semcA, semnA, semwA, sem8A,
                       semcB, semnB, semwB, sem8B):
    core = lax.axis_index("c")
    sub = lax.axis_index("s")
    wid = sub * 2 + core

    lane = lax.iota(jnp.int32, _NCH)
    lane3 = [lane * 3 + j for j in range(3)]
    npairs = jnp.where(wid < (_NPAIR % _NW), _NPAIR // _NW + 1, _NPAIR // _NW)

    chunks = (
        (curA, candA, nbrA, clkA, cidxA, idx0A, w8A, semcA, semnA, semwA, sem8A),
        (curB, candB, nbrB, clkB, cidxB, idx0B, w8B, semcB, semnB, semwB, sem8B),
    )

    def issue(ch, t, idx):
        (_, cand_v, nbr_v, _, cidx_v, idx0_v, _, sem_c, sem_n, _, _) = ch
        valid = idx > 0
        idx0 = jnp.clip(idx, 1, _N) - 1
        idx0_v[...] = idx0
        for j in range(3):
            plsc.store_scatter(cidx_v, [lane3[j]], idx0 * 3 + j)
        ccand = pltpu.async_copy(table.at[idx0_v], cand_v.at[t % 2], sem_c)
        cnbr = pltpu.async_copy(nbr.at[cidx_v], nbr_v, sem_n)
        return (ccand, cnbr, valid)

    def step(ch, t, base, idx, inflight, pend):
        (cur_v, cand_v, nbr_v, _, _, _, w8_v, _, _, sem_w, _) = ch
        ccand, cnbr, valid = inflight
        ccand.wait()
        cnbr.wait()
        buf = t % 2
        wr = pltpu.make_async_copy(
            cand_v.at[buf], x3_out.at[t, pl.ds(base, _NCH), :], sem_w)
        wr.start()
        pend.append(wr)

        bsel = jnp.full((_NCH,), buf, jnp.int32)
        off = [jnp.full((_NCH,), j * _D, jnp.int32) for j in range(3)]

        def dbody(i, carry):
            s0, s1, s2 = carry
            for u in range(_UNROLL):
                dcol = jnp.full((_NCH,), i * _UNROLL + u, jnp.int32)
                cu = plsc.load_gather(cur_v, [lane, dcol])
                c0 = plsc.load_gather(cand_v, [bsel, lane, off[0] + dcol])
                c1 = plsc.load_gather(cand_v, [bsel, lane, off[1] + dcol])
                c2 = plsc.load_gather(cand_v, [bsel, lane, off[2] + dcol])
                s0 = s0 + c0 * cu
                s1 = s1 + c1 * cu
                s2 = s2 + c2 * cu
            return s0, s1, s2

        zero = jnp.zeros((_NCH,), jnp.float32)
        s0, s1, s2 = lax.fori_loop(0, _D // _UNROLL, dbody, (zero, zero, zero))
        s0 = jnp.where(valid, s0, 0.0)
        s1 = jnp.where(valid, s1, 0.0)
        s2 = jnp.where(valid, s2, 0.0)
        m01 = jnp.maximum(s0, s1)
        maxv = jnp.maximum(m01, s2)
        mi = jnp.where(s1 > s0, 1, 0)
        mi = jnp.where(s2 > m01, 2, mi)
        nz = maxv != 0.0
        nxt = plsc.load_gather(nbr_v, [lane * 3 + mi])
        idx = jnp.where(nz, nxt, idx)
        tfull = jnp.full((_NCH,), t, jnp.int32)
        for j in range(3):
            wv = jnp.where((mi == j) & nz, 1.0, 0.0)
            plsc.store_scatter(w8_v, [tfull, lane, jnp.full((_NCH,), j, jnp.int32)], wv)
        return idx, pend

    def pair_body(pp, _):
        baseA = (wid + pp * _NW) * 2 * _NCH
        baseB = baseA + _NCH
        pltpu.sync_copy(click.at[pl.ds(baseA, _NCH)], clkA)
        pltpu.sync_copy(news.at[pl.ds(baseA, _NCH), :], curA)
        pltpu.sync_copy(click.at[pl.ds(baseB, _NCH)], clkB)
        pltpu.sync_copy(news.at[pl.ds(baseB, _NCH), :], curB)
        idxA = clkA[...]
        idxB = clkB[...]

        infA = issue(chunks[0], 0, idxA)
        infB = issue(chunks[1], 0, idxB)
        pendA, pendB = [], []
        for t in range(_L):
            idxA, pendA = step(chunks[0], t, baseA, idxA, infA, pendA)
            if t + 1 < _L:
                while len(pendA) > 1:
                    pendA.pop(0).wait()
                infA = issue(chunks[0], t + 1, idxA)
            idxB, pendB = step(chunks[1], t, baseB, idxB, infB, pendB)
            if t + 1 < _L:
                while len(pendB) > 1:
                    pendB.pop(0).wait()
                infB = issue(chunks[1], t + 1, idxB)

        w8wA = pltpu.make_async_copy(w8A, w8_out.at[:, pl.ds(baseA, _NCH), :], sem8A)
        w8wA.start()
        w8wB = pltpu.make_async_copy(w8B, w8_out.at[:, pl.ds(baseB, _NCH), :], sem8B)
        w8wB.start()
        for p in pendA:
            p.wait()
        for p in pendB:
            p.wait()
        w8wA.wait()
        w8wB.wait()
        return 0

    lax.fori_loop(0, npairs, pair_body, 0)


def _chunk_scratches():
    return [
        pltpu.VMEM((_NCH, _D), jnp.float32),
        pltpu.VMEM((2, _NCH, 3 * _D), jnp.float32),
        pltpu.VMEM((_NCH * 3,), jnp.int32),
        pltpu.VMEM((_NCH,), jnp.int32),
        pltpu.VMEM((_NCH * 3,), jnp.int32),
        pltpu.VMEM((_NCH,), jnp.int32),
        pltpu.VMEM((_L, _NCH, 8), jnp.float32),
    ]


def _make_chain_kernel():
    mesh = plsc.VectorSubcoreMesh(core_axis_name="c", subcore_axis_name="s")
    return pl.kernel(
        _chain_kernel_body,
        out_type=[
            jax.ShapeDtypeStruct((_L, _CHAINS, 3 * _D), jnp.float32),
            jax.ShapeDtypeStruct((_L, _CHAINS, 8), jnp.float32),
        ],
        mesh=mesh,
        scratch_types=(
            _chunk_scratches() + _chunk_scratches()
            + [pltpu.SemaphoreType.DMA] * 8
        ),
        compiler_params=pltpu.CompilerParams(needs_layout_passes=False),
    )


def _encoder_body(c_ref, w_ref,
                  wq_ref, bq_ref, wk_ref, bk_ref, wv_ref, bv_ref,
                  g1_ref, b1l_ref, mw1_ref, mb1_ref, mw2_ref, mb2_ref,
                  g2_ref, b2l_ref, out_ref, o_scr):
    f32 = jnp.float32
    S = _HIS * _L
    G = _BB * _HIS
    scale = 1.0 / jnp.sqrt(jnp.float32(_DK))
    riota = lax.broadcasted_iota(jnp.int32, (_HIS, S), 0)
    ciota = lax.broadcasted_iota(jnp.int32, (_HIS, S), 1)
    seg = (ciota % _HIS) == riota

    xsel = []
    for t in range(_L):
        ct = c_ref[t]
        wt = w_ref[t]
        acc = ct[:, 0:_D] * wt[:, 0:1]
        acc = acc + ct[:, _D:2 * _D] * wt[:, 1:2]
        acc = acc + ct[:, 2 * _D:3 * _D] * wt[:, 2:3]
        xsel.append(acc)

    outs = []
    for sub in range(_BB):
        xm = jnp.concatenate(
            [xsel[t][sub * _HIS:(sub + 1) * _HIS] for t in range(_L)], axis=0)
        q = jnp.dot(xm, wq_ref[...], preferred_element_type=f32) + bq_ref[...]
        k = jnp.dot(xm, wk_ref[...], preferred_element_type=f32) + bk_ref[...]
        v = jnp.dot(xm, wv_ref[...], preferred_element_type=f32) + bv_ref[...]
        for h in range(_H):
            sl = slice(h * _DK, (h + 1) * _DK)
            qh, kh, vh = q[:, sl], k[:, sl], v[:, sl]
            s = lax.dot_general(qh, kh, (((1,), (1,)), ((), ())),
                                preferred_element_type=f32) * scale
            s = s - jnp.max(s, axis=-1, keepdims=True)
            p = jnp.exp(s)
            p = p / jnp.sum(p, axis=-1, keepdims=True)
            o_scr[:, sl] = jnp.dot(p, vh, preferred_element_type=f32)
        o = o_scr[...]
        mu = jnp.mean(o, axis=-1, keepdims=True)
        xc = o - mu
        var = jnp.mean(xc * xc, axis=-1, keepdims=True)
        x1 = xc / jnp.sqrt(var + 1e-5) * g1_ref[...] + b1l_ref[...]
        t1 = jnp.tanh(jnp.dot(x1, mw1_ref[...], preferred_element_type=f32)
                      + mb1_ref[...])
        e = jnp.sum(t1 * mw2_ref[...], axis=1, keepdims=True) + mb2_ref[...]
        eT = jnp.transpose(e)
        sm = jnp.where(seg, jnp.broadcast_to(eT, (_HIS, S)), -1e30)
        mg = jnp.max(sm, axis=1, keepdims=True)
        P = jnp.where(seg, jnp.exp(sm - mg), 0.0)
        Wm = P / jnp.sum(P, axis=1, keepdims=True)
        pooled = jnp.dot(Wm, x1, preferred_element_type=f32)
        mu2 = jnp.mean(pooled, axis=-1, keepdims=True)
        pc = pooled - mu2
        var2 = jnp.mean(pc * pc, axis=-1, keepdims=True)
        outs.append(pc / jnp.sqrt(var2 + 1e-5) * g2_ref[...] + b2l_ref[...])
    out_ref[...] = jnp.concatenate(outs, axis=0)


def _full(shape):
    return pl.BlockSpec(shape, lambda b: tuple(0 for _ in shape))


def kernel(news_input, click_history, outputs_dict, neighbors, Wq, bq, Wk, bk,
           Wv, bv, ln1_g, ln1_b, w1, b1, w2, b2, ln2_g, ln2_b):
    click = click_history.reshape(_CHAINS)
    news = news_input.reshape(_CHAINS, _D)
    table = outputs_dict.reshape(_N, 3 * _D)
    nbr_flat = neighbors.reshape(_N * 3)

    x3, w8 = _make_chain_kernel()(table, click, nbr_flat, news)
    G = _BB * _HIS

    out2 = pl.pallas_call(
        _encoder_body,
        grid=(_B // _BB,),
        in_specs=(
            [
                pl.BlockSpec((_L, G, 3 * _D), lambda b: (0, b, 0)),
                pl.BlockSpec((_L, G, 8), lambda b: (0, b, 0)),
                _full((_D, _D)), _full((1, _D)),
                _full((_D, _D)), _full((1, _D)),
                _full((_D, _D)), _full((1, _D)),
                _full((1, _D)), _full((1, _D)),
                _full((_D, _HID)), _full((1, _HID)),
                _full((1, _HID)), _full((1, 1)),
                _full((1, _D)), _full((1, _D)),
            ]
        ),
        out_specs=pl.BlockSpec((G, _D), lambda b: (b, 0)),
        out_shape=jax.ShapeDtypeStruct((_CHAINS, _D), jnp.float32),
        scratch_shapes=[pltpu.VMEM((_HIS * _L, _D), jnp.float32)],
    )(
        x3, w8,
        Wq, bq.reshape(1, _D), Wk, bk.reshape(1, _D), Wv, bv.reshape(1, _D),
        ln1_g.reshape(1, _D), ln1_b.reshape(1, _D),
        w1, b1.reshape(1, _HID), w2.reshape(1, _HID), b2.reshape(1, 1),
        ln2_g.reshape(1, _D), ln2_b.reshape(1, _D),
    )
    return out2.reshape(_B, _HIS, _D)


# speedup vs baseline: 1.2224x; 1.2224x over previous
import jax
import jax.numpy as jnp
from jax import lax
from jax.experimental import pallas as pl
from jax.experimental.pallas import tpu as pltpu
from jax.experimental.pallas import tpu_sc as plsc

_B, _HIS, _D = 64, 50, 256
_N = 100000
_L = 6
_H, _DK = 16, 16
_HID = 200

_NW = 32
_NCH = 16
_CHAINS = _B * _HIS
_NCHUNK = _CHAINS // _NCH
_NPAIR = _NCHUNK // 2
_UNROLL = 4
_BB = 8


def _chain_kernel_body(table, click, nbr, news, x3_out, w8_out,
                       curA, candA, nbrA, clkA, cidxA, idx0A, w8A, s3A,
                       curB, candB, nbrB, clkB, cidxB, idx0B, w8B, s3B,
                       semcA, semnA, semwA, sem8A,
                       semcB, semnB, semwB, sem8B):
    core = lax.axis_index("c")
    sub = lax.axis_index("s")
    wid = sub * 2 + core

    lane = lax.iota(jnp.int32, _NCH)
    lane3 = [lane * 3 + j for j in range(3)]
    npairs = jnp.where(wid < (_NPAIR % _NW), _NPAIR // _NW + 1, _NPAIR // _NW)

    chunks = (
        (curA, candA, nbrA, clkA, cidxA, idx0A, w8A, s3A,
         semcA, semnA, semwA, sem8A),
        (curB, candB, nbrB, clkB, cidxB, idx0B, w8B, s3B,
         semcB, semnB, semwB, sem8B),
    )

    def issue(ch, t, idx):
        (_, cand_v, nbr_v, _, cidx_v, idx0_v, _, _, sem_c, sem_n, _, _) = ch
        valid = idx > 0
        idx0 = jnp.clip(idx, 1, _N) - 1
        idx0_v[...] = idx0
        for j in range(3):
            plsc.store_scatter(cidx_v, [lane3[j]], idx0 * 3 + j)
        ccand = pltpu.async_copy(table.at[idx0_v], cand_v.at[t % 2], sem_c)
        cnbr = pltpu.async_copy(nbr.at[cidx_v], nbr_v, sem_n)
        return (ccand, cnbr, valid)

    def step(ch, t, base, idx, inflight, pend):
        (cur_v, cand_v, nbr_v, _, _, _, w8_v, s3_v, _, _, sem_w, _) = ch
        ccand, cnbr, valid = inflight
        ccand.wait()
        cnbr.wait()
        buf = t % 2
        wr = pltpu.make_async_copy(
            cand_v.at[buf], x3_out.at[t, pl.ds(base, _NCH), :], sem_w)
        wr.start()
        pend.append(wr)

        def cbody(c, _):
            zero = jnp.zeros((_NCH,), jnp.float32)
            a0, a1, a2 = zero, zero, zero
            for b in range(_D // _NCH):
                cu = cur_v[c, pl.ds(b * _NCH, _NCH)]
                a0 = a0 + cand_v[buf, c, pl.ds(b * _NCH, _NCH)] * cu
                a1 = a1 + cand_v[buf, c, pl.ds(_D + b * _NCH, _NCH)] * cu
                a2 = a2 + cand_v[buf, c, pl.ds(2 * _D + b * _NCH, _NCH)] * cu
            s3_v[0, c, pl.ds(0, _NCH)] = a0
            s3_v[1, c, pl.ds(0, _NCH)] = a1
            s3_v[2, c, pl.ds(0, _NCH)] = a2
            return 0

        lax.fori_loop(0, _NCH, cbody, 0)
        sums = []
        for j in range(3):
            jf = jnp.full((_NCH,), j, jnp.int32)
            r = jnp.zeros((_NCH,), jnp.float32)
            for l in range(_NCH):
                r = r + plsc.load_gather(s3_v, [jf, lane, jnp.full((_NCH,), l, jnp.int32)])
            sums.append(r)
        s0, s1, s2 = sums
        s0 = jnp.where(valid, s0, 0.0)
        s1 = jnp.where(valid, s1, 0.0)
        s2 = jnp.where(valid, s2, 0.0)
        m01 = jnp.maximum(s0, s1)
        maxv = jnp.maximum(m01, s2)
        mi = jnp.where(s1 > s0, 1, 0)
        mi = jnp.where(s2 > m01, 2, mi)
        nz = maxv != 0.0
        nxt = plsc.load_gather(nbr_v, [lane * 3 + mi])
        idx = jnp.where(nz, nxt, idx)
        tfull = jnp.full((_NCH,), t, jnp.int32)
        for j in range(3):
            wv = jnp.where((mi == j) & nz, 1.0, 0.0)
            plsc.store_scatter(w8_v, [tfull, lane, jnp.full((_NCH,), j, jnp.int32)], wv)
        return idx, pend

    def pair_body(pp, _):
        baseA = (wid + pp * _NW) * 2 * _NCH
        baseB = baseA + _NCH
        pltpu.sync_copy(click.at[pl.ds(baseA, _NCH)], clkA)
        pltpu.sync_copy(news.at[pl.ds(baseA, _NCH), :], curA)
        pltpu.sync_copy(click.at[pl.ds(baseB, _NCH)], clkB)
        pltpu.sync_copy(news.at[pl.ds(baseB, _NCH), :], curB)
        idxA = clkA[...]
        idxB = clkB[...]

        infA = issue(chunks[0], 0, idxA)
        infB = issue(chunks[1], 0, idxB)
        pendA, pendB = [], []
        for t in range(_L):
            idxA, pendA = step(chunks[0], t, baseA, idxA, infA, pendA)
            if t + 1 < _L:
                while len(pendA) > 1:
                    pendA.pop(0).wait()
                infA = issue(chunks[0], t + 1, idxA)
            idxB, pendB = step(chunks[1], t, baseB, idxB, infB, pendB)
            if t + 1 < _L:
                while len(pendB) > 1:
                    pendB.pop(0).wait()
                infB = issue(chunks[1], t + 1, idxB)

        w8wA = pltpu.make_async_copy(w8A, w8_out.at[:, pl.ds(baseA, _NCH), :], sem8A)
        w8wA.start()
        w8wB = pltpu.make_async_copy(w8B, w8_out.at[:, pl.ds(baseB, _NCH), :], sem8B)
        w8wB.start()
        for p in pendA:
            p.wait()
        for p in pendB:
            p.wait()
        w8wA.wait()
        w8wB.wait()
        return 0

    lax.fori_loop(0, npairs, pair_body, 0)


def _chunk_scratches():
    return [
        pltpu.VMEM((_NCH, _D), jnp.float32),
        pltpu.VMEM((2, _NCH, 3 * _D), jnp.float32),
        pltpu.VMEM((_NCH * 3,), jnp.int32),
        pltpu.VMEM((_NCH,), jnp.int32),
        pltpu.VMEM((_NCH * 3,), jnp.int32),
        pltpu.VMEM((_NCH,), jnp.int32),
        pltpu.VMEM((_L, _NCH, 8), jnp.float32),
        pltpu.VMEM((3, _NCH, 17), jnp.float32),
    ]


def _make_chain_kernel():
    mesh = plsc.VectorSubcoreMesh(core_axis_name="c", subcore_axis_name="s")
    return pl.kernel(
        _chain_kernel_body,
        out_type=[
            jax.ShapeDtypeStruct((_L, _CHAINS, 3 * _D), jnp.float32),
            jax.ShapeDtypeStruct((_L, _CHAINS, 8), jnp.float32),
        ],
        mesh=mesh,
        scratch_types=(
            _chunk_scratches() + _chunk_scratches()
            + [pltpu.SemaphoreType.DMA] * 8
        ),
        compiler_params=pltpu.CompilerParams(needs_layout_passes=False),
    )


def _encoder_body(c_ref, w_ref,
                  wq_ref, bq_ref, wk_ref, bk_ref, wv_ref, bv_ref,
                  g1_ref, b1l_ref, mw1_ref, mb1_ref, mw2_ref, mb2_ref,
                  g2_ref, b2l_ref, out_ref, o_scr):
    f32 = jnp.float32
    S = _HIS * _L
    G = _BB * _HIS
    scale = 1.0 / jnp.sqrt(jnp.float32(_DK))
    riota = lax.broadcasted_iota(jnp.int32, (_HIS, S), 0)
    ciota = lax.broadcasted_iota(jnp.int32, (_HIS, S), 1)
    seg = (ciota % _HIS) == riota

    xsel = []
    for t in range(_L):
        ct = c_ref[t]
        wt = w_ref[t]
        acc = ct[:, 0:_D] * wt[:, 0:1]
        acc = acc + ct[:, _D:2 * _D] * wt[:, 1:2]
        acc = acc + ct[:, 2 * _D:3 * _D] * wt[:, 2:3]
        xsel.append(acc)

    outs = []
    for sub in range(_BB):
        xm = jnp.concatenate(
            [xsel[t][sub * _HIS:(sub + 1) * _HIS] for t in range(_L)], axis=0)
        q = jnp.dot(xm, wq_ref[...], preferred_element_type=f32) + bq_ref[...]
        k = jnp.dot(xm, wk_ref[...], preferred_element_type=f32) + bk_ref[...]
        v = jnp.dot(xm, wv_ref[...], preferred_element_type=f32) + bv_ref[...]
        for h in range(_H):
            sl = slice(h * _DK, (h + 1) * _DK)
            qh, kh, vh = q[:, sl], k[:, sl], v[:, sl]
            s = lax.dot_general(qh, kh, (((1,), (1,)), ((), ())),
                                preferred_element_type=f32) * scale
            s = s - jnp.max(s, axis=-1, keepdims=True)
            p = jnp.exp(s)
            p = p / jnp.sum(p, axis=-1, keepdims=True)
            o_scr[:, sl] = jnp.dot(p, vh, preferred_element_type=f32)
        o = o_scr[...]
        mu = jnp.mean(o, axis=-1, keepdims=True)
        xc = o - mu
        var = jnp.mean(xc * xc, axis=-1, keepdims=True)
        x1 = xc / jnp.sqrt(var + 1e-5) * g1_ref[...] + b1l_ref[...]
        t1 = jnp.tanh(jnp.dot(x1, mw1_ref[...], preferred_element_type=f32)
                      + mb1_ref[...])
        e = jnp.sum(t1 * mw2_ref[...], axis=1, keepdims=True) + mb2_ref[...]
        eT = jnp.transpose(e)
        sm = jnp.where(seg, jnp.broadcast_to(eT, (_HIS, S)), -1e30)
        mg = jnp.max(sm, axis=1, keepdims=True)
        P = jnp.where(seg, jnp.exp(sm - mg), 0.0)
        Wm = P / jnp.sum(P, axis=1, keepdims=True)
        pooled = jnp.dot(Wm, x1, preferred_element_type=f32)
        mu2 = jnp.mean(pooled, axis=-1, keepdims=True)
        pc = pooled - mu2
        var2 = jnp.mean(pc * pc, axis=-1, keepdims=True)
        outs.append(pc / jnp.sqrt(var2 + 1e-5) * g2_ref[...] + b2l_ref[...])
    out_ref[...] = jnp.concatenate(outs, axis=0)


def _full(shape):
    return pl.BlockSpec(shape, lambda b: tuple(0 for _ in shape))


def kernel(news_input, click_history, outputs_dict, neighbors, Wq, bq, Wk, bk,
           Wv, bv, ln1_g, ln1_b, w1, b1, w2, b2, ln2_g, ln2_b):
    click = click_history.reshape(_CHAINS)
    news = news_input.reshape(_CHAINS, _D)
    table = outputs_dict.reshape(_N, 3 * _D)
    nbr_flat = neighbors.reshape(_N * 3)

    x3, w8 = _make_chain_kernel()(table, click, nbr_flat, news)
    G = _BB * _HIS

    out2 = pl.pallas_call(
        _encoder_body,
        grid=(_B // _BB,),
        in_specs=(
            [
                pl.BlockSpec((_L, G, 3 * _D), lambda b: (0, b, 0)),
                pl.BlockSpec((_L, G, 8), lambda b: (0, b, 0)),
                _full((_D, _D)), _full((1, _D)),
                _full((_D, _D)), _full((1, _D)),
                _full((_D, _D)), _full((1, _D)),
                _full((1, _D)), _full((1, _D)),
                _full((_D, _HID)), _full((1, _HID)),
                _full((1, _HID)), _full((1, 1)),
                _full((1, _D)), _full((1, _D)),
            ]
        ),
        out_specs=pl.BlockSpec((G, _D), lambda b: (b, 0)),
        out_shape=jax.ShapeDtypeStruct((_CHAINS, _D), jnp.float32),
        scratch_shapes=[pltpu.VMEM((_HIS * _L, _D), jnp.float32)],
    )(
        x3, w8,
        Wq, bq.reshape(1, _D), Wk, bk.reshape(1, _D), Wv, bv.reshape(1, _D),
        ln1_g.reshape(1, _D), ln1_b.reshape(1, _D),
        w1, b1.reshape(1, _HID), w2.reshape(1, _HID), b2.reshape(1, 1),
        ln2_g.reshape(1, _D), ln2_b.reshape(1, _D),
    )
    return out2.reshape(_B, _HIS, _D)


# speedup vs baseline: 1.2277x; 1.0044x over previous
import jax
import jax.numpy as jnp
from jax import lax
from jax.experimental import pallas as pl
from jax.experimental.pallas import tpu as pltpu
from jax.experimental.pallas import tpu_sc as plsc

_B, _HIS, _D = 64, 50, 256
_N = 100000
_L = 6
_H, _DK = 16, 16
_HID = 200

_NW = 32
_NCH = 16
_CHAINS = _B * _HIS
_NCHUNK = _CHAINS // _NCH
_NPAIR = _NCHUNK // 2
_UNROLL = 4
_BB = 8


def _chain_kernel_body(table, click, nbr, news, x3_out, w8_out,
                       curA, candA, nbrA, clkA, cidxA, idx0A, w8A, s3A,
                       curB, candB, nbrB, clkB, cidxB, idx0B, w8B, s3B,
                       semcA, semnA, semwA, sem8A,
                       semcB, semnB, semwB, sem8B):
    core = lax.axis_index("c")
    sub = lax.axis_index("s")
    wid = sub * 2 + core

    lane = lax.iota(jnp.int32, _NCH)
    lane3 = [lane * 3 + j for j in range(3)]
    npairs = jnp.where(wid < (_NPAIR % _NW), _NPAIR // _NW + 1, _NPAIR // _NW)

    chunks = (
        (curA, candA, nbrA, clkA, cidxA, idx0A, w8A, s3A,
         semcA, semnA, semwA, sem8A),
        (curB, candB, nbrB, clkB, cidxB, idx0B, w8B, s3B,
         semcB, semnB, semwB, sem8B),
    )

    def issue(ch, t, idx):
        (_, cand_v, nbr_v, _, cidx_v, idx0_v, _, _, sem_c, sem_n, _, _) = ch
        valid = idx > 0
        idx0 = jnp.clip(idx, 1, _N) - 1
        idx0_v[...] = idx0
        for j in range(3):
            plsc.store_scatter(cidx_v, [lane3[j]], idx0 * 3 + j)
        ccand = pltpu.async_copy(table.at[idx0_v], cand_v.at[t % 2], sem_c)
        cnbr = pltpu.async_copy(nbr.at[cidx_v], nbr_v, sem_n)
        return (ccand, cnbr, valid)

    def step(ch, t, base, idx, inflight, pend):
        (cur_v, cand_v, nbr_v, _, _, _, w8_v, s3_v, _, _, sem_w, _) = ch
        ccand, cnbr, valid = inflight
        ccand.wait()
        cnbr.wait()
        buf = t % 2
        wr = pltpu.make_async_copy(
            cand_v.at[buf], x3_out.at[t, pl.ds(base, _NCH), :], sem_w)
        wr.start()
        pend.append(wr)

        def cbody(c, _):
            zero = jnp.zeros((_NCH,), jnp.float32)
            a0, a1, a2 = zero, zero, zero
            for b in range(_D // _NCH):
                cu = cur_v[c, pl.ds(b * _NCH, _NCH)]
                a0 = a0 + cand_v[buf, c, pl.ds(b * _NCH, _NCH)] * cu
                a1 = a1 + cand_v[buf, c, pl.ds(_D + b * _NCH, _NCH)] * cu
                a2 = a2 + cand_v[buf, c, pl.ds(2 * _D + b * _NCH, _NCH)] * cu
            s3_v[0, c, pl.ds(0, _NCH)] = a0
            s3_v[1, c, pl.ds(0, _NCH)] = a1
            s3_v[2, c, pl.ds(0, _NCH)] = a2
            return 0

        lax.fori_loop(0, _NCH, cbody, 0)
        sums = []
        for j in range(3):
            jf = jnp.full((_NCH,), j, jnp.int32)
            r = jnp.zeros((_NCH,), jnp.float32)
            for l in range(_NCH):
                r = r + plsc.load_gather(s3_v, [jf, lane, jnp.full((_NCH,), l, jnp.int32)])
            sums.append(r)
        s0, s1, s2 = sums
        s0 = jnp.where(valid, s0, 0.0)
        s1 = jnp.where(valid, s1, 0.0)
        s2 = jnp.where(valid, s2, 0.0)
        m01 = jnp.maximum(s0, s1)
        maxv = jnp.maximum(m01, s2)
        mi = jnp.where(s1 > s0, 1, 0)
        mi = jnp.where(s2 > m01, 2, mi)
        nz = maxv != 0.0
        nxt = plsc.load_gather(nbr_v, [lane * 3 + mi])
        idx = jnp.where(nz, nxt, idx)
        tfull = jnp.full((_NCH,), t, jnp.int32)
        for j in range(3):
            wv = jnp.where((mi == j) & nz, 1.0, 0.0)
            plsc.store_scatter(w8_v, [tfull, lane, jnp.full((_NCH,), j, jnp.int32)], wv)
        return idx, pend

    def pair_body(pp, _):
        baseA = (wid + pp * _NW) * 2 * _NCH
        baseB = baseA + _NCH
        pltpu.sync_copy(click.at[pl.ds(baseA, _NCH)], clkA)
        pltpu.sync_copy(news.at[pl.ds(baseA, _NCH), :], curA)
        pltpu.sync_copy(click.at[pl.ds(baseB, _NCH)], clkB)
        pltpu.sync_copy(news.at[pl.ds(baseB, _NCH), :], curB)
        idxA = clkA[...]
        idxB = clkB[...]

        infA = issue(chunks[0], 0, idxA)
        infB = issue(chunks[1], 0, idxB)
        pendA, pendB = [], []
        for t in range(_L):
            idxA, pendA = step(chunks[0], t, baseA, idxA, infA, pendA)
            if t + 1 < _L:
                while len(pendA) > 1:
                    pendA.pop(0).wait()
                infA = issue(chunks[0], t + 1, idxA)
            idxB, pendB = step(chunks[1], t, baseB, idxB, infB, pendB)
            if t + 1 < _L:
                while len(pendB) > 1:
                    pendB.pop(0).wait()
                infB = issue(chunks[1], t + 1, idxB)

        w8wA = pltpu.make_async_copy(w8A, w8_out.at[:, pl.ds(baseA, _NCH), :], sem8A)
        w8wA.start()
        w8wB = pltpu.make_async_copy(w8B, w8_out.at[:, pl.ds(baseB, _NCH), :], sem8B)
        w8wB.start()
        for p in pendA:
            p.wait()
        for p in pendB:
            p.wait()
        w8wA.wait()
        w8wB.wait()
        return 0

    lax.fori_loop(0, npairs, pair_body, 0)


def _chunk_scratches():
    return [
        pltpu.VMEM((_NCH, _D), jnp.float32),
        pltpu.VMEM((2, _NCH, 3 * _D), jnp.float32),
        pltpu.VMEM((_NCH * 3,), jnp.int32),
        pltpu.VMEM((_NCH,), jnp.int32),
        pltpu.VMEM((_NCH * 3,), jnp.int32),
        pltpu.VMEM((_NCH,), jnp.int32),
        pltpu.VMEM((_L, _NCH, 8), jnp.float32),
        pltpu.VMEM((3, _NCH, 17), jnp.float32),
    ]


def _make_chain_kernel():
    mesh = plsc.VectorSubcoreMesh(core_axis_name="c", subcore_axis_name="s")
    return pl.kernel(
        _chain_kernel_body,
        out_type=[
            jax.ShapeDtypeStruct((_L, _CHAINS, 3 * _D), jnp.float32),
            jax.ShapeDtypeStruct((_L, _CHAINS, 8), jnp.float32),
        ],
        mesh=mesh,
        scratch_types=(
            _chunk_scratches() + _chunk_scratches()
            + [pltpu.SemaphoreType.DMA] * 8
        ),
        compiler_params=pltpu.CompilerParams(needs_layout_passes=False),
    )


def _encoder_body(c_ref, w_ref,
                  wq_ref, bq_ref, wk_ref, bk_ref, wv_ref, bv_ref,
                  g1_ref, b1l_ref, mw1_ref, mb1_ref, mw2_ref, mb2_ref,
                  g2_ref, b2l_ref, out_ref, o_scr):
    f32 = jnp.float32
    S = _HIS * _L
    G = _BB * _HIS
    scale = 1.0 / jnp.sqrt(jnp.float32(_DK))
    riota = lax.broadcasted_iota(jnp.int32, (_HIS, S), 0)
    ciota = lax.broadcasted_iota(jnp.int32, (_HIS, S), 1)
    seg = (ciota % _HIS) == riota

    xsel = []
    for t in range(_L):
        ct = c_ref[t]
        wt = w_ref[t]
        acc = ct[:, 0:_D] * wt[:, 0:1]
        acc = acc + ct[:, _D:2 * _D] * wt[:, 1:2]
        acc = acc + ct[:, 2 * _D:3 * _D] * wt[:, 2:3]
        xsel.append(acc)

    outs = []
    for sub in range(_BB):
        xm = jnp.concatenate(
            [xsel[t][sub * _HIS:(sub + 1) * _HIS] for t in range(_L)], axis=0)
        xmb = xm.astype(jnp.bfloat16)
        q = jnp.dot(xmb, wq_ref[...], preferred_element_type=f32) + bq_ref[...]
        k = jnp.dot(xmb, wk_ref[...], preferred_element_type=f32) + bk_ref[...]
        v = jnp.dot(xmb, wv_ref[...], preferred_element_type=f32) + bv_ref[...]
        qb = q.astype(jnp.bfloat16)
        kb = k.astype(jnp.bfloat16)
        vb = v.astype(jnp.bfloat16)
        for h in range(_H):
            sl = slice(h * _DK, (h + 1) * _DK)
            qh, kh, vh = qb[:, sl], kb[:, sl], vb[:, sl]
            s = lax.dot_general(qh, kh, (((1,), (1,)), ((), ())),
                                preferred_element_type=f32) * scale
            s = s - jnp.max(s, axis=-1, keepdims=True)
            p = jnp.exp(s)
            p = p / jnp.sum(p, axis=-1, keepdims=True)
            o_scr[:, sl] = jnp.dot(p.astype(jnp.bfloat16), vh,
                                   preferred_element_type=f32)
        o = o_scr[...]
        mu = jnp.mean(o, axis=-1, keepdims=True)
        xc = o - mu
        var = jnp.mean(xc * xc, axis=-1, keepdims=True)
        x1 = xc / jnp.sqrt(var + 1e-5) * g1_ref[...] + b1l_ref[...]
        t1 = jnp.tanh(jnp.dot(x1, mw1_ref[...], preferred_element_type=f32)
                      + mb1_ref[...])
        e = jnp.sum(t1 * mw2_ref[...], axis=1, keepdims=True) + mb2_ref[...]
        eT = jnp.transpose(e)
        sm = jnp.where(seg, jnp.broadcast_to(eT, (_HIS, S)), -1e30)
        mg = jnp.max(sm, axis=1, keepdims=True)
        P = jnp.where(seg, jnp.exp(sm - mg), 0.0)
        Wm = P / jnp.sum(P, axis=1, keepdims=True)
        pooled = jnp.dot(Wm, x1, preferred_element_type=f32)
        mu2 = jnp.mean(pooled, axis=-1, keepdims=True)
        pc = pooled - mu2
        var2 = jnp.mean(pc * pc, axis=-1, keepdims=True)
        outs.append(pc / jnp.sqrt(var2 + 1e-5) * g2_ref[...] + b2l_ref[...])
    out_ref[...] = jnp.concatenate(outs, axis=0)


def _full(shape):
    return pl.BlockSpec(shape, lambda b: tuple(0 for _ in shape))


def kernel(news_input, click_history, outputs_dict, neighbors, Wq, bq, Wk, bk,
           Wv, bv, ln1_g, ln1_b, w1, b1, w2, b2, ln2_g, ln2_b):
    click = click_history.reshape(_CHAINS)
    news = news_input.reshape(_CHAINS, _D)
    table = outputs_dict.reshape(_N, 3 * _D)
    nbr_flat = neighbors.reshape(_N * 3)

    x3, w8 = _make_chain_kernel()(table, click, nbr_flat, news)
    G = _BB * _HIS

    out2 = pl.pallas_call(
        _encoder_body,
        grid=(_B // _BB,),
        in_specs=(
            [
                pl.BlockSpec((_L, G, 3 * _D), lambda b: (0, b, 0)),
                pl.BlockSpec((_L, G, 8), lambda b: (0, b, 0)),
                _full((_D, _D)), _full((1, _D)),
                _full((_D, _D)), _full((1, _D)),
                _full((_D, _D)), _full((1, _D)),
                _full((1, _D)), _full((1, _D)),
                _full((_D, _HID)), _full((1, _HID)),
                _full((1, _HID)), _full((1, 1)),
                _full((1, _D)), _full((1, _D)),
            ]
        ),
        out_specs=pl.BlockSpec((G, _D), lambda b: (b, 0)),
        out_shape=jax.ShapeDtypeStruct((_CHAINS, _D), jnp.float32),
        scratch_shapes=[pltpu.VMEM((_HIS * _L, _D), jnp.float32)],
    )(
        x3, w8,
        Wq.astype(jnp.bfloat16), bq.reshape(1, _D),
        Wk.astype(jnp.bfloat16), bk.reshape(1, _D),
        Wv.astype(jnp.bfloat16), bv.reshape(1, _D),
        ln1_g.reshape(1, _D), ln1_b.reshape(1, _D),
        w1, b1.reshape(1, _HID), w2.reshape(1, _HID), b2.reshape(1, 1),
        ln2_g.reshape(1, _D), ln2_b.reshape(1, _D),
    )
    return out2.reshape(_B, _HIS, _D)
